# baseline (speedup 1.0000x reference)
import jax
import jax.numpy as jnp
from jax.experimental import pallas as pl
from jax.experimental.pallas import tpu as pltpu

LEADS = 16384
L = 4096
NUM_MASK = L // 2
ROWS_PER_BLK = 512


def _body(x_ref, s_ref, o_ref):
    x = x_ref[...]
    s = s_ref[...]
    idx = jax.lax.broadcasted_iota(jnp.int32, (1, L), 1)
    off = (idx - s) & (L - 1)
    o_ref[...] = jnp.where(off < NUM_MASK, jnp.zeros((), x.dtype), x)


def kernel(x, starts):
    leads, length = x.shape
    s2 = starts.reshape(leads, 1)
    grid = leads // ROWS_PER_BLK
    return pl.pallas_call(
        _body,
        grid=(grid,),
        in_specs=[
            pl.BlockSpec((ROWS_PER_BLK, length), lambda i: (i, 0)),
            pl.BlockSpec((ROWS_PER_BLK, 1), lambda i: (i, 0)),
        ],
        out_specs=pl.BlockSpec((ROWS_PER_BLK, length), lambda i: (i, 0)),
        out_shape=jax.ShapeDtypeStruct((leads, length), x.dtype),
    )(x, s2)
